# baseline (speedup 1.0000x reference)
import functools

import jax
import jax.numpy as jnp
from jax import lax
from jax.experimental import pallas as pl
from jax.experimental.pallas import tpu as pltpu
from jax.experimental.pallas import tpu_sc as plsc

NC = 2
NS = 16
NW = NC * NS
LANES = 16

N_PLAYERS = 1000000
EMBED_DIM = 32
BATCH = 16384

C = BATCH // NW
GCHUNK = 128
NG = C // GCHUNK
NM = C // LANES


def _body(player_hbm, pos_emb_hbm, idx_hbm, pos_idx_hbm, out_hbm,
          idx_v, pos_idx_v, pos_v, rows_v, sem):
    wid = lax.axis_index("s") * NC + lax.axis_index("c")

    pltpu.sync_copy(idx_hbm.at[wid], idx_v)
    pltpu.sync_copy(pos_idx_hbm.at[wid], pos_idx_v)
    pltpu.sync_copy(pos_emb_hbm, pos_v)

    copies = []
    for t in range(NG):
        copies.append(pltpu.async_copy(
            player_hbm.at[idx_v.at[t]],
            rows_v.at[pl.ds(t * GCHUNK, GCHUNK)],
            sem,
        ))

    acc = jnp.zeros((LANES,), jnp.int32)
    for t in range(NG):
        for k in range(GCHUNK // LANES):
            idx16 = idx_v[t, pl.ds(k * LANES, LANES)]
            acc = acc | jnp.where(idx16 == 0, 1, 0)
    any_zero = plsc.all_reduce_population_count(acc != 0)[0] > 0

    for cp in copies:
        cp.wait()

    @pl.when(any_zero)
    def _fixup():
        lane = lax.iota(jnp.int32, 16)
        for t in range(NG):
            for k in range(GCHUNK // LANES):
                idx16 = idx_v[t, pl.ds(k * LANES, LANES)]
                m = idx16 == 0

                @pl.when(plsc.all_reduce_population_count(m)[0] > 0)
                def _chunk(t=t, k=k, m=m):
                    base = t * GCHUNK + k * LANES
                    p16 = pos_idx_v[pl.ds(base, LANES)]
                    b16 = base + lane

                    def jbody(j, carry):
                        j16 = jnp.full((LANES,), j, jnp.int32)
                        vals = plsc.load_gather(pos_v, [p16, j16])
                        plsc.store_scatter(rows_v, [b16, j16], vals, mask=m)
                        return carry

                    lax.fori_loop(0, EMBED_DIM, jbody, 0)

    pltpu.sync_copy(rows_v, out_hbm.at[wid])


@jax.jit
def _run(player_emb, position_emb, idx2, pos_idx2):
    mesh = plsc.VectorSubcoreMesh(core_axis_name="c", subcore_axis_name="s")
    return pl.kernel(
        _body,
        out_type=jax.ShapeDtypeStruct((NW, C, EMBED_DIM), jnp.float32),
        mesh=mesh,
        compiler_params=pltpu.CompilerParams(
            needs_layout_passes=False, use_tc_tiling_on_sc=False),
        scratch_types=[
            pltpu.VMEM((NG, GCHUNK), jnp.int32),
            pltpu.VMEM((C,), jnp.int32),
            pltpu.VMEM((2, EMBED_DIM), jnp.float32),
            pltpu.VMEM((C, EMBED_DIM), jnp.float32),
            pltpu.SemaphoreType.DMA,
        ],
    )(player_emb, position_emb, idx2, pos_idx2)


def kernel(player_emb, position_emb, player_indices, position_indices):
    idx2 = player_indices.astype(jnp.int32).reshape(NW, NG, GCHUNK)
    pos_idx2 = position_indices.astype(jnp.int32).reshape(NW, C)
    out = _run(player_emb, position_emb, idx2, pos_idx2)
    return out.reshape(BATCH, EMBED_DIM)


# speedup vs baseline: 1.0009x; 1.0009x over previous
import jax
import jax.numpy as jnp
from jax import lax
from jax.experimental import pallas as pl
from jax.experimental.pallas import tpu as pltpu
from jax.experimental.pallas import tpu_sc as plsc

NC = 2
NS = 16
NW = NC * NS
LANES = 16

N_PLAYERS = 1000000
EMBED_DIM = 32
BATCH = 16384

C = BATCH // NW
GCHUNK = 128
NG = C // GCHUNK


def _body(player_hbm, pos_emb_hbm, idx_hbm, pos_idx_hbm, out_hbm,
          idx_v, pos_idx_v, pos_v, rows_v, sem):
    wid = lax.axis_index("s") * NC + lax.axis_index("c")
    base = wid * C

    pltpu.sync_copy(idx_hbm.at[pl.ds(base, C)], idx_v)
    pltpu.sync_copy(pos_idx_hbm.at[pl.ds(base, C)], pos_idx_v)
    pltpu.sync_copy(pos_emb_hbm, pos_v)

    copies = []
    for t in range(NG):
        copies.append(pltpu.async_copy(
            player_hbm.at[idx_v.at[pl.ds(t * GCHUNK, GCHUNK)]],
            rows_v.at[pl.ds(t * GCHUNK, GCHUNK)],
            sem,
        ))

    acc = jnp.zeros((LANES,), jnp.int32)
    for k in range(C // LANES):
        idx16 = idx_v[pl.ds(k * LANES, LANES)]
        acc = acc | jnp.where(idx16 == 0, 1, 0)
    any_zero = plsc.all_reduce_population_count(acc != 0)[0] > 0

    for cp in copies:
        cp.wait()

    @pl.when(any_zero)
    def _fixup():
        lane = lax.iota(jnp.int32, 16)
        for k in range(C // LANES):
            idx16 = idx_v[pl.ds(k * LANES, LANES)]
            m = idx16 == 0

            @pl.when(plsc.all_reduce_population_count(m)[0] > 0)
            def _chunk(k=k, m=m):
                cbase = k * LANES
                p16 = pos_idx_v[pl.ds(cbase, LANES)]
                b16 = cbase + lane

                def jbody(j, carry):
                    j16 = jnp.full((LANES,), j, jnp.int32)
                    vals = plsc.load_gather(pos_v, [p16, j16])
                    plsc.store_scatter(rows_v, [b16, j16], vals, mask=m)
                    return carry

                lax.fori_loop(0, EMBED_DIM, jbody, 0)

    pltpu.sync_copy(rows_v, out_hbm.at[pl.ds(base, C)])


@jax.jit
def _run(player_emb, position_emb, player_indices, position_indices):
    mesh = plsc.VectorSubcoreMesh(core_axis_name="c", subcore_axis_name="s")
    return pl.kernel(
        _body,
        out_type=jax.ShapeDtypeStruct((BATCH, EMBED_DIM), jnp.float32),
        mesh=mesh,
        compiler_params=pltpu.CompilerParams(
            needs_layout_passes=False, use_tc_tiling_on_sc=False),
        scratch_types=[
            pltpu.VMEM((C,), jnp.int32),
            pltpu.VMEM((C,), jnp.int32),
            pltpu.VMEM((2, EMBED_DIM), jnp.float32),
            pltpu.VMEM((C, EMBED_DIM), jnp.float32),
            pltpu.SemaphoreType.DMA,
        ],
    )(player_emb, position_emb, player_indices, position_indices)


def kernel(player_emb, position_emb, player_indices, position_indices):
    return _run(player_emb, position_emb,
                player_indices.astype(jnp.int32),
                position_indices.astype(jnp.int32))


# speedup vs baseline: 2.5187x; 2.5164x over previous
import jax
import jax.numpy as jnp
from jax import lax
from jax.experimental import pallas as pl
from jax.experimental.pallas import tpu as pltpu
from jax.experimental.pallas import tpu_sc as plsc

NC = 2
NS = 16
NW = NC * NS
LANES = 16

N_PLAYERS = 1000000
EMBED_DIM = 32
BATCH = 16384

C = BATCH // NW
RND = 64
NR = C // RND


def _body(player_hbm, pos_emb_hbm, idx_hbm, pos_idx_hbm, out_hbm,
          idx_v, gidx_v, pos_idx_v, pos_v, blocks_v, rows_v, sem):
    wid = lax.axis_index("s") * NC + lax.axis_index("c")
    base = wid * C

    pltpu.sync_copy(idx_hbm.at[pl.ds(base, C)], idx_v)
    pltpu.sync_copy(pos_idx_hbm.at[pl.ds(base, C)], pos_idx_v)
    pltpu.sync_copy(pos_emb_hbm, pos_v)

    acc = jnp.zeros((LANES,), jnp.int32)
    for k in range(C // LANES):
        idx16 = idx_v[pl.ds(k * LANES, LANES)]
        gidx_v[pl.ds(k * LANES, LANES)] = idx16 >> 3
        acc = acc | jnp.where(idx16 == 0, 1, 0)
    any_zero = plsc.all_reduce_population_count(acc != 0)[0] > 0

    lane = lax.iota(jnp.int32, 16)

    for rd in range(NR):
        rbase = rd * RND

        copies = []
        for c in range(RND // LANES):
            cb = rbase + c * LANES
            idx16 = idx_v[pl.ds(cb, LANES)]
            g16 = idx16 >> 3
            r16 = idx16 & 7
            for l in range(LANES):
                copies.append(pltpu.async_copy(
                    player_hbm.at[g16[l], r16[l]],
                    rows_v.at[c * LANES + l],
                    sem,
                ))
        for cp in copies:
            cp.wait()

        @pl.when(any_zero)
        def _fixup(rbase=rbase):
            for c in range(RND // LANES):
                cb = rbase + c * LANES
                idx16 = idx_v[pl.ds(cb, LANES)]
                m = idx16 == 0

                @pl.when(plsc.all_reduce_population_count(m)[0] > 0)
                def _chunk(c=c, cb=cb, m=m):
                    p16 = pos_idx_v[pl.ds(cb, LANES)]
                    b16 = c * LANES + lane

                    def jbody(j, carry):
                        j16 = jnp.full((LANES,), j, jnp.int32)
                        vals = plsc.load_gather(pos_v, [p16, j16])
                        plsc.store_scatter(rows_v, [b16, j16], vals, mask=m)
                        return carry

                    lax.fori_loop(0, EMBED_DIM, jbody, 0)

        pltpu.sync_copy(rows_v, out_hbm.at[pl.ds(base + rbase, RND)])


@jax.jit
def _run(player3, position_emb, player_indices, position_indices):
    mesh = plsc.VectorSubcoreMesh(core_axis_name="c", subcore_axis_name="s")
    return pl.kernel(
        _body,
        out_type=jax.ShapeDtypeStruct((BATCH, EMBED_DIM), jnp.float32),
        mesh=mesh,
        compiler_params=pltpu.CompilerParams(
            needs_layout_passes=False, use_tc_tiling_on_sc=True),
        scratch_types=[
            pltpu.VMEM((C,), jnp.int32),
            pltpu.VMEM((C,), jnp.int32),
            pltpu.VMEM((C,), jnp.int32),
            pltpu.VMEM((2, EMBED_DIM), jnp.float32),
            pltpu.VMEM((RND, 8, EMBED_DIM), jnp.float32),
            pltpu.VMEM((RND, EMBED_DIM), jnp.float32),
            pltpu.SemaphoreType.DMA,
        ],
    )(player3, position_emb, player_indices, position_indices)


def kernel(player_emb, position_emb, player_indices, position_indices):
    player3 = player_emb.reshape(N_PLAYERS // 8, 8, EMBED_DIM)
    return _run(player3, position_emb,
                player_indices.astype(jnp.int32),
                position_indices.astype(jnp.int32))


# speedup vs baseline: 8.6148x; 3.4204x over previous
import jax
import jax.numpy as jnp
from jax import lax
from jax.experimental import pallas as pl
from jax.experimental.pallas import tpu as pltpu
from jax.experimental.pallas import tpu_sc as plsc

NC = 2
NS = 16
NW = NC * NS
LANES = 16

N_PLAYERS = 1000000
EMBED_DIM = 32
DB = EMBED_DIM // 8
BATCH = 16384

C = BATCH // NW
RND = 64
NR = C // RND


def _body(player_hbm, pos_emb_hbm, idx_hbm, pos_idx_hbm, out_hbm,
          idx_v, r_v, pos_idx_v, pos_v, stage0, stage1, col_v,
          sem0, sem1):
    wid = lax.axis_index("s") * NC + lax.axis_index("c")
    base = wid * C
    stages = [stage0, stage1]
    sems = [sem0, sem1]

    pltpu.sync_copy(idx_hbm.at[pl.ds(base, C)], idx_v)

    def fire(g):
        rbase = g * RND
        stage = stages[g % 2]
        sem = sems[g % 2]

        @pl.loop(0, RND)
        def _f(i):
            v = idx_v[pl.ds(rbase + i, LANES)]
            p0 = pl.multiple_of((v[0] >> 4) << 4, LANES)
            pltpu.async_copy(
                player_hbm.at[:, :, pl.ds(p0, LANES)],
                stage.at[:, :, pl.ds(pl.multiple_of(i * LANES, LANES), LANES)],
                sem,
            )

    fire(0)

    pltpu.sync_copy(pos_idx_hbm.at[pl.ds(base, C)], pos_idx_v)
    pltpu.sync_copy(pos_emb_hbm, pos_v)

    acc = jnp.zeros((LANES,), jnp.int32)
    for k in range(C // LANES):
        idx16 = idx_v[pl.ds(k * LANES, LANES)]
        r_v[pl.ds(k * LANES, LANES)] = idx16 & 15
        acc = acc | jnp.where(idx16 == 0, 1, 0)
    any_zero = plsc.all_reduce_population_count(acc != 0)[0] > 0

    lane = lax.iota(jnp.int32, 16)

    for g in range(NR):
        if g + 1 < NR:
            fire(g + 1)

        pltpu.make_async_copy(
            out_hbm.at[:, :, pl.ds(0, RND * LANES)], stages[g % 2],
            sems[g % 2]).wait()

        rbase = g * RND
        stage = stages[g % 2]
        for c in range(RND // LANES):
            cb = rbase + c * LANES
            r16 = r_v[pl.ds(cb, LANES)]
            slot16 = (c * LANES + lane) * LANES + r16
            i16 = cb + lane

            def ebody(d, carry, r16=r16, slot16=slot16, i16=i16):
                vals = plsc.load_gather(
                    stage,
                    [jnp.full((LANES,), d >> 3, jnp.int32),
                     jnp.full((LANES,), d & 7, jnp.int32),
                     slot16])
                plsc.store_scatter(
                    col_v,
                    [jnp.full((LANES,), d >> 3, jnp.int32),
                     jnp.full((LANES,), d & 7, jnp.int32),
                     i16],
                    vals)
                return carry

            lax.fori_loop(0, EMBED_DIM, ebody, 0)

    @pl.when(any_zero)
    def _fixup():
        for k in range(C // LANES):
            cb = k * LANES
            idx16 = idx_v[pl.ds(cb, LANES)]
            m = idx16 == 0

            @pl.when(plsc.all_reduce_population_count(m)[0] > 0)
            def _chunk(cb=cb, m=m):
                p16 = pos_idx_v[pl.ds(cb, LANES)]
                i16 = cb + lane

                def jbody(d, carry):
                    d16 = jnp.full((LANES,), d, jnp.int32)
                    vals = plsc.load_gather(pos_v, [p16, d16])
                    plsc.store_scatter(
                        col_v,
                        [jnp.full((LANES,), d >> 3, jnp.int32),
                         jnp.full((LANES,), d & 7, jnp.int32),
                         i16],
                        vals, mask=m)
                    return carry

                lax.fori_loop(0, EMBED_DIM, jbody, 0)

    pltpu.sync_copy(col_v, out_hbm.at[:, :, pl.ds(base, C)])


@jax.jit
def _run(playerT, position_emb, player_indices, position_indices):
    mesh = plsc.VectorSubcoreMesh(core_axis_name="c", subcore_axis_name="s")
    outT = pl.kernel(
        _body,
        out_type=jax.ShapeDtypeStruct((DB, 8, BATCH), jnp.float32),
        mesh=mesh,
        compiler_params=pltpu.CompilerParams(
            needs_layout_passes=False, use_tc_tiling_on_sc=True),
        scratch_types=[
            pltpu.VMEM((C,), jnp.int32),
            pltpu.VMEM((C,), jnp.int32),
            pltpu.VMEM((C,), jnp.int32),
            pltpu.VMEM((2, EMBED_DIM), jnp.float32),
            pltpu.VMEM((DB, 8, RND * LANES), jnp.float32),
            pltpu.VMEM((DB, 8, RND * LANES), jnp.float32),
            pltpu.VMEM((DB, 8, C), jnp.float32),
            pltpu.SemaphoreType.DMA,
            pltpu.SemaphoreType.DMA,
        ],
    )(playerT, position_emb, player_indices, position_indices)
    return outT.reshape(EMBED_DIM, BATCH).T


def kernel(player_emb, position_emb, player_indices, position_indices):
    playerT = player_emb.T.reshape(DB, 8, N_PLAYERS)
    return _run(playerT, position_emb,
                player_indices.astype(jnp.int32),
                position_indices.astype(jnp.int32))


# speedup vs baseline: 8.8490x; 1.0272x over previous
import jax
import jax.numpy as jnp
from jax import lax
from jax.experimental import pallas as pl
from jax.experimental.pallas import tpu as pltpu
from jax.experimental.pallas import tpu_sc as plsc

NC = 2
NS = 16
NW = NC * NS
LANES = 16

N_PLAYERS = 1000000
EMBED_DIM = 32
DB = EMBED_DIM // 8
BATCH = 16384

C = BATCH // NW
RND = 64
NR = C // RND


def _body(player_hbm, pos_emb_hbm, idx_hbm, pos_idx_hbm, out_hbm,
          idx_v, r_v, pos_idx_v, pos_v, stage0, stage1, col_v,
          sem0, sem1):
    wid = lax.axis_index("s") * NC + lax.axis_index("c")
    base = wid * C
    stages = [stage0, stage1]
    sems = [sem0, sem1]

    pltpu.sync_copy(idx_hbm.at[pl.ds(base, C)], idx_v)

    def fire(g):
        rbase = g * RND
        stage = stages[g % 2]
        sem = sems[g % 2]

        @pl.loop(0, RND, step=4)
        def _f(i):
            v = idx_v[pl.ds(rbase + i, LANES)]
            for u in range(4):
                p0 = pl.multiple_of((v[u] >> 4) << 4, LANES)
                pltpu.async_copy(
                    player_hbm.at[:, :, pl.ds(p0, LANES)],
                    stage.at[:, :, pl.ds(
                        pl.multiple_of((i + u) * LANES, LANES), LANES)],
                    sem,
                )

    fire(0)

    pltpu.sync_copy(pos_idx_hbm.at[pl.ds(base, C)], pos_idx_v)
    pltpu.sync_copy(pos_emb_hbm, pos_v)

    acc = jnp.zeros((LANES,), jnp.int32)
    for k in range(C // LANES):
        idx16 = idx_v[pl.ds(k * LANES, LANES)]
        r_v[pl.ds(k * LANES, LANES)] = idx16 & 15
        acc = acc | jnp.where(idx16 == 0, 1, 0)
    any_zero = plsc.all_reduce_population_count(acc != 0)[0] > 0

    lane = lax.iota(jnp.int32, 16)

    for g in range(NR):
        if g + 1 < NR:
            fire(g + 1)

        pltpu.make_async_copy(
            out_hbm.at[:, :, pl.ds(0, RND * LANES)], stages[g % 2],
            sems[g % 2]).wait()

        rbase = g * RND
        stage = stages[g % 2]
        for c in range(RND // LANES):
            cb = rbase + c * LANES
            r16 = r_v[pl.ds(cb, LANES)]
            slot16 = (c * LANES + lane) * LANES + r16
            i16 = cb + lane

            def ebody(d, carry, r16=r16, slot16=slot16, i16=i16):
                vals = plsc.load_gather(
                    stage,
                    [jnp.full((LANES,), d >> 3, jnp.int32),
                     jnp.full((LANES,), d & 7, jnp.int32),
                     slot16])
                plsc.store_scatter(
                    col_v,
                    [jnp.full((LANES,), d >> 3, jnp.int32),
                     jnp.full((LANES,), d & 7, jnp.int32),
                     i16],
                    vals)
                return carry

            lax.fori_loop(0, EMBED_DIM, ebody, 0)

    @pl.when(any_zero)
    def _fixup():
        for k in range(C // LANES):
            cb = k * LANES
            idx16 = idx_v[pl.ds(cb, LANES)]
            m = idx16 == 0

            @pl.when(plsc.all_reduce_population_count(m)[0] > 0)
            def _chunk(cb=cb, m=m):
                p16 = pos_idx_v[pl.ds(cb, LANES)]
                i16 = cb + lane

                def jbody(d, carry):
                    d16 = jnp.full((LANES,), d, jnp.int32)
                    vals = plsc.load_gather(pos_v, [p16, d16])
                    plsc.store_scatter(
                        col_v,
                        [jnp.full((LANES,), d >> 3, jnp.int32),
                         jnp.full((LANES,), d & 7, jnp.int32),
                         i16],
                        vals, mask=m)
                    return carry

                lax.fori_loop(0, EMBED_DIM, jbody, 0)

    pltpu.sync_copy(col_v, out_hbm.at[:, :, pl.ds(base, C)])


@jax.jit
def _run(playerT, position_emb, player_indices, position_indices):
    mesh = plsc.VectorSubcoreMesh(core_axis_name="c", subcore_axis_name="s")
    outT = pl.kernel(
        _body,
        out_type=jax.ShapeDtypeStruct((DB, 8, BATCH), jnp.float32),
        mesh=mesh,
        compiler_params=pltpu.CompilerParams(
            needs_layout_passes=False, use_tc_tiling_on_sc=True),
        scratch_types=[
            pltpu.VMEM((C,), jnp.int32),
            pltpu.VMEM((C,), jnp.int32),
            pltpu.VMEM((C,), jnp.int32),
            pltpu.VMEM((2, EMBED_DIM), jnp.float32),
            pltpu.VMEM((DB, 8, RND * LANES), jnp.float32),
            pltpu.VMEM((DB, 8, RND * LANES), jnp.float32),
            pltpu.VMEM((DB, 8, C), jnp.float32),
            pltpu.SemaphoreType.DMA,
            pltpu.SemaphoreType.DMA,
        ],
    )(playerT, position_emb, player_indices, position_indices)
    return outT.reshape(EMBED_DIM, BATCH).T


def kernel(player_emb, position_emb, player_indices, position_indices):
    playerT = player_emb.T.reshape(DB, 8, N_PLAYERS)
    return _run(playerT, position_emb,
                player_indices.astype(jnp.int32),
                position_indices.astype(jnp.int32))


# speedup vs baseline: 8.8531x; 1.0005x over previous
import jax
import jax.numpy as jnp
from jax import lax
from jax.experimental import pallas as pl
from jax.experimental.pallas import tpu as pltpu
from jax.experimental.pallas import tpu_sc as plsc

NC = 2
NS = 16
NW = NC * NS
LANES = 16

N_PLAYERS = 1000000
EMBED_DIM = 32
DB = EMBED_DIM // 8
BATCH = 16384

C = BATCH // NW
RND = 64
NR = C // RND


def _body(player_hbm, pos_emb_hbm, idx_hbm, pos_idx_hbm, out_hbm,
          idx_v, r_v, pos_idx_v, pos_v, stage0, stage1, col_v,
          sem0, sem1):
    wid = lax.axis_index("s") * NC + lax.axis_index("c")
    base = wid * C
    stages = [stage0, stage1]
    sems = [sem0, sem1]

    pltpu.sync_copy(idx_hbm.at[pl.ds(base, C)], idx_v.at[pl.ds(0, C)])

    def fire(g):
        rbase = g * RND
        stage = stages[g % 2]
        sem = sems[g % 2]

        @pl.loop(0, RND, step=4)
        def _f(i):
            v = idx_v[pl.ds(rbase + i, LANES)]
            for u in range(4):
                p0 = pl.multiple_of((v[u] >> 4) << 4, LANES)
                pltpu.async_copy(
                    player_hbm.at[:, :, pl.ds(p0, LANES)],
                    stage.at[:, :, pl.ds(
                        pl.multiple_of((i + u) * LANES, LANES), LANES)],
                    sem,
                )

    fire(0)

    pltpu.sync_copy(pos_idx_hbm.at[pl.ds(base, C)], pos_idx_v)
    pltpu.sync_copy(pos_emb_hbm, pos_v)

    acc = jnp.zeros((LANES,), jnp.int32)
    for k in range(C // LANES):
        idx16 = idx_v[pl.ds(k * LANES, LANES)]
        r_v[pl.ds(k * LANES, LANES)] = idx16 & 15
        acc = acc | jnp.where(idx16 == 0, 1, 0)
    any_zero = plsc.all_reduce_population_count(acc != 0)[0] > 0

    lane = lax.iota(jnp.int32, 16)

    for g in range(NR):
        if g + 1 < NR:
            fire(g + 1)

        pltpu.make_async_copy(
            out_hbm.at[:, :, pl.ds(0, RND * LANES)], stages[g % 2],
            sems[g % 2]).wait()

        rbase = g * RND
        stage = stages[g % 2]
        for c in range(RND // LANES):
            cb = rbase + c * LANES
            r16 = r_v[pl.ds(cb, LANES)]
            slot16 = (c * LANES + lane) * LANES + r16
            i16 = cb + lane

            def ebody(d, carry, r16=r16, slot16=slot16, i16=i16):
                vals = plsc.load_gather(
                    stage,
                    [jnp.full((LANES,), d >> 3, jnp.int32),
                     jnp.full((LANES,), d & 7, jnp.int32),
                     slot16])
                plsc.store_scatter(
                    col_v,
                    [jnp.full((LANES,), d >> 3, jnp.int32),
                     jnp.full((LANES,), d & 7, jnp.int32),
                     i16],
                    vals)
                return carry

            lax.fori_loop(0, EMBED_DIM, ebody, 0)

    @pl.when(any_zero)
    def _fixup():
        for k in range(C // LANES):
            cb = k * LANES
            idx16 = idx_v[pl.ds(cb, LANES)]
            m = idx16 == 0

            @pl.when(plsc.all_reduce_population_count(m)[0] > 0)
            def _chunk(cb=cb, m=m):
                p16 = pos_idx_v[pl.ds(cb, LANES)]
                i16 = cb + lane

                def jbody(d, carry):
                    d16 = jnp.full((LANES,), d, jnp.int32)
                    vals = plsc.load_gather(pos_v, [p16, d16])
                    plsc.store_scatter(
                        col_v,
                        [jnp.full((LANES,), d >> 3, jnp.int32),
                         jnp.full((LANES,), d & 7, jnp.int32),
                         i16],
                        vals, mask=m)
                    return carry

                lax.fori_loop(0, EMBED_DIM, jbody, 0)

    pltpu.sync_copy(col_v, out_hbm.at[:, :, pl.ds(base, C)])


@jax.jit
def _run(playerT, position_emb, player_indices, position_indices):
    mesh = plsc.VectorSubcoreMesh(core_axis_name="c", subcore_axis_name="s")
    outT = pl.kernel(
        _body,
        out_type=jax.ShapeDtypeStruct((DB, 8, BATCH), jnp.float32),
        mesh=mesh,
        compiler_params=pltpu.CompilerParams(
            needs_layout_passes=False, use_tc_tiling_on_sc=True),
        scratch_types=[
            pltpu.VMEM((C + LANES,), jnp.int32),
            pltpu.VMEM((C,), jnp.int32),
            pltpu.VMEM((C,), jnp.int32),
            pltpu.VMEM((2, EMBED_DIM), jnp.float32),
            pltpu.VMEM((DB, 8, RND * LANES), jnp.float32),
            pltpu.VMEM((DB, 8, RND * LANES), jnp.float32),
            pltpu.VMEM((DB, 8, C), jnp.float32),
            pltpu.SemaphoreType.DMA,
            pltpu.SemaphoreType.DMA,
        ],
    )(playerT, position_emb, player_indices, position_indices)
    return outT.reshape(EMBED_DIM, BATCH).T


def kernel(player_emb, position_emb, player_indices, position_indices):
    playerT = player_emb.T.reshape(DB, 8, N_PLAYERS)
    return _run(playerT, position_emb,
                player_indices.astype(jnp.int32),
                position_indices.astype(jnp.int32))
